# Initial kernel scaffold; baseline (speedup 1.0000x reference)
import jax
import jax.numpy as jnp
from jax.experimental import pallas as pl


def kernel(user_emb, item_emb, edge_src, edge_dst, edge_weight):
    raise NotImplementedError("write your pallas kernel here")


# speedup vs baseline: 1.4958x; 1.4958x over previous
import dataclasses

import jax
import jax.numpy as jnp
from jax import lax
from jax.experimental import pallas as pl
from jax.experimental.pallas import tpu as pltpu
from jax.experimental.pallas import tpu_sc as plsc

NUM_USERS = 30000
NUM_ITEMS = 20000
N_NODES = NUM_USERS + NUM_ITEMS
D = 64
DH = 32
E = 800000
N_LAYERS = 3
NS = 16
LANES = 16
SUB = 128
GSUB = 4
CHUNK = SUB * GSUB
RPT = 392
NCHUNK = RPT // GSUB
E_PAD = RPT * NS * SUB
N_PAD = 50048
STRIPE = N_PAD // NS
ZBF = 136
NZB = STRIPE // ZBF


def _sc_body(emb0, src2d, dst2d, w2d, out, tbuf,
             accum, idx_s, idx_d, wbuf, rows, zbuf):
    c = lax.axis_index("core")
    t = lax.axis_index("subcore")
    iota16 = lax.iota(jnp.int32, LANES)
    zeros16 = jnp.zeros((LANES,), jnp.float32)
    coff = jnp.full((LANES,), 0, jnp.int32) + c * N_PAD

    @pl.loop(0, ZBF)
    def _(i):
        zbuf[i, pl.ds(0, LANES)] = zeros16
        zbuf[i, pl.ds(LANES, LANES)] = zeros16

    def zero_accum():
        for b in range(NZB):
            pltpu.sync_copy(zbuf, accum.at[pl.ds(t * STRIPE + b * ZBF, ZBF)])

    def layer(gather_src):
        rbase0 = t * RPT

        @pl.loop(0, NCHUNK)
        def _(ci):
            rbase = rbase0 + ci * GSUB
            pltpu.sync_copy(src2d.at[pl.ds(rbase, GSUB)], idx_s)
            pltpu.sync_copy(dst2d.at[pl.ds(rbase, GSUB)], idx_d)
            pltpu.sync_copy(w2d.at[pl.ds(rbase, GSUB)], wbuf)

            @pl.loop(0, GSUB)
            def _(si):
                for g in range(SUB // LANES):
                    sl = pl.ds(g * LANES, LANES)
                    idx_s[si, sl] = idx_s[si, sl] + coff

            @pl.loop(0, GSUB)
            def _(s):
                rslice = pl.ds(s * SUB, SUB)
                pltpu.sync_copy(gather_src.at[idx_s.at[s]], rows.at[rslice])
                for g in range(SUB // LANES):
                    w16 = wbuf[s, pl.ds(g * LANES, LANES)]
                    ridx = iota16 + (s * SUB + g * LANES)
                    for d in range(DH):
                        cidx = jnp.full((LANES,), d, jnp.int32)
                        col = plsc.load_gather(rows, [ridx, cidx])
                        plsc.store_scatter(rows, [ridx, cidx], col * w16)
                pltpu.sync_copy(rows.at[rslice], accum.at[idx_d.at[s]],
                                add=True)

    for l in range(N_LAYERS):
        zero_accum()
        plsc.subcore_barrier()
        layer(emb0 if l == 0 else tbuf.at[l - 1])
        plsc.subcore_barrier()
        if l < N_LAYERS - 1:
            for b in range(NZB):
                off = t * STRIPE + b * ZBF
                pltpu.sync_copy(accum.at[pl.ds(off, ZBF)],
                                tbuf.at[l].at[pl.ds(c * N_PAD + off, ZBF)])
            plsc.subcore_barrier()

    for b in range(NZB):
        a0 = t * STRIPE + b * ZBF
        g0 = c * N_PAD + a0
        pltpu.sync_copy(emb0.at[pl.ds(g0, ZBF)], rows.at[pl.ds(0, ZBF)])
        pltpu.sync_copy(tbuf.at[0].at[pl.ds(g0, ZBF)],
                        rows.at[pl.ds(ZBF, ZBF)])
        pltpu.sync_copy(tbuf.at[1].at[pl.ds(g0, ZBF)],
                        rows.at[pl.ds(2 * ZBF, ZBF)])
        pltpu.sync_copy(accum.at[pl.ds(a0, ZBF)], rows.at[pl.ds(3 * ZBF, ZBF)])

        @pl.loop(0, ZBF)
        def _(i):
            for j in range(DH // LANES):
                sl = pl.ds(j * LANES, LANES)
                acc = ((rows[i, sl] + rows[ZBF + i, sl])
                       + (rows[2 * ZBF + i, sl] + rows[3 * ZBF + i, sl]))
                zbuf[i, sl] = acc * 0.25
        pltpu.sync_copy(zbuf, out.at[pl.ds(g0, ZBF)])


@jax.jit
def _run(emb_halves, src2d, dst2d, w2d):
    cp = pltpu.CompilerParams(use_tc_tiling_on_sc=False)
    if "needs_layout_passes" in pltpu.CompilerParams.__dataclass_fields__:
        cp = dataclasses.replace(cp, needs_layout_passes=False)
    mesh = plsc.VectorSubcoreMesh(core_axis_name="core",
                                  subcore_axis_name="subcore")
    f = pl.kernel(
        _sc_body,
        out_type=(
            jax.ShapeDtypeStruct((2 * N_PAD, DH), jnp.float32),
            jax.ShapeDtypeStruct((N_LAYERS - 1, 2 * N_PAD, DH), jnp.float32),
        ),
        mesh=mesh,
        scratch_types=[
            pltpu.VMEM_SHARED((N_PAD, DH), jnp.float32),
            pltpu.VMEM((GSUB, SUB), jnp.int32),
            pltpu.VMEM((GSUB, SUB), jnp.int32),
            pltpu.VMEM((GSUB, SUB), jnp.float32),
            pltpu.VMEM((4 * ZBF, DH), jnp.float32),
            pltpu.VMEM((ZBF, DH), jnp.float32),
        ],
        compiler_params=cp,
    )
    return f(emb_halves, src2d, dst2d, w2d)


def kernel(user_emb, item_emb, edge_src, edge_dst, edge_weight):
    all_emb = jnp.concatenate([user_emb, item_emb], axis=0)
    pad_n = jnp.zeros((N_PAD - N_NODES, DH), jnp.float32)
    emb_halves = jnp.concatenate(
        [all_emb[:, :DH], pad_n, all_emb[:, DH:], pad_n], axis=0)
    n_extra = E_PAD - E
    pad_idx = (jnp.arange(n_extra, dtype=jnp.int32) * 8) % N_NODES
    src_p = jnp.concatenate([edge_src, pad_idx])
    dst_p = jnp.concatenate([edge_dst, pad_idx])
    w_p = jnp.concatenate([edge_weight, jnp.zeros((n_extra,), jnp.float32)])
    src2d = src_p.reshape(E_PAD // SUB, SUB)
    dst2d = dst_p.reshape(E_PAD // SUB, SUB)
    w2d = w_p.reshape(E_PAD // SUB, SUB)
    out_sum, _ = _run(emb_halves, src2d, dst2d, w2d)
    light = jnp.concatenate(
        [out_sum[:N_NODES], out_sum[N_PAD:N_PAD + N_NODES]], axis=1)
    return light[:NUM_USERS], light[NUM_USERS:]
